# Initial kernel scaffold; baseline (speedup 1.0000x reference)
import jax
import jax.numpy as jnp
from jax.experimental import pallas as pl


def kernel(x, gate_w, w1, b1, w2, b2):
    raise NotImplementedError("write your pallas kernel here")


# speedup vs baseline: 1.3450x; 1.3450x over previous
import functools

import jax
import jax.numpy as jnp
from jax import lax
from jax.experimental import pallas as pl
from jax.experimental.pallas import tpu as pltpu

S = 2048
D_MODEL = 768
D_FF = 3072
E = 8
CHUNK = 256
NF = 2
FFB = D_FF // NF


def _moe_dense_body(x_ref, cw_ref, w1_ref, b1_ref, w2_ref, b2_ref, out_ref):
    e = pl.program_id(0)
    f = pl.program_id(1)

    @pl.when(jnp.logical_and(e == 0, f == 0))
    def _init():
        out_ref[...] = jnp.zeros_like(out_ref)

    w1 = w1_ref[0].astype(jnp.bfloat16)
    w2 = w2_ref[0].astype(jnp.bfloat16)
    b1 = b1_ref[0, 0]
    b2 = jnp.where(f == 0, b2_ref[0, 0], 0.0)

    def chunk_step(c, _):
        xb = x_ref[pl.ds(c * CHUNK, CHUNK), :]
        h = lax.dot_general(xb, w1, (((1,), (1,)), ((), ())),
                            preferred_element_type=jnp.float32)
        h = jnp.maximum(h + b1[None, :], 0.0).astype(jnp.bfloat16)
        o = lax.dot_general(h, w2, (((1,), (1,)), ((), ())),
                            preferred_element_type=jnp.float32)
        o = o + b2[None, :]
        cw = cw_ref[0, 0, pl.ds(c * CHUNK, CHUNK)]
        out_ref[pl.ds(c * CHUNK, CHUNK), :] += o * cw[:, None]
        return 0

    lax.fori_loop(0, S // CHUNK, chunk_step, 0)


@jax.jit
def _moe_dense(x_bf16, cw, w1, b1, w2, b2):
    return pl.pallas_call(
        _moe_dense_body,
        grid=(E, NF),
        in_specs=[
            pl.BlockSpec((S, D_MODEL), lambda e, f: (0, 0)),
            pl.BlockSpec((1, 1, S), lambda e, f: (e, 0, 0)),
            pl.BlockSpec((1, FFB, D_MODEL), lambda e, f: (e, f, 0)),
            pl.BlockSpec((1, 1, FFB), lambda e, f: (e, 0, f)),
            pl.BlockSpec((1, D_MODEL, FFB), lambda e, f: (e, 0, f)),
            pl.BlockSpec((1, 1, D_MODEL), lambda e, f: (e, 0, 0)),
        ],
        out_specs=pl.BlockSpec((S, D_MODEL), lambda e, f: (0, 0)),
        out_shape=jax.ShapeDtypeStruct((S, D_MODEL), jnp.float32),
        compiler_params=pltpu.CompilerParams(
            dimension_semantics=("arbitrary", "arbitrary"),
        ),
    )(x_bf16, cw, w1, b1, w2, b2)


def kernel(x, gate_w, w1, b1, w2, b2):
    s, b, h = x.shape
    x_flat = x.reshape(s * b, h)

    logits = x_flat @ gate_w.T
    probs = jax.nn.softmax(logits, axis=-1)
    topk_probs, topk_idx = jax.lax.top_k(probs, 2)

    oh0 = jax.nn.one_hot(topk_idx[:, 0], E, dtype=jnp.float32)
    oh1 = jax.nn.one_hot(topk_idx[:, 1], E, dtype=jnp.float32)
    cw = oh0 * topk_probs[:, 0:1] + oh1 * topk_probs[:, 1:2]

    y_flat = _moe_dense(
        x_flat.astype(jnp.bfloat16),
        cw.T.reshape(E, 1, s * b),
        w1,
        b1.reshape(E, 1, D_FF),
        w2,
        b2.reshape(E, 1, D_MODEL),
    )
    return y_flat.reshape(s, b, h)
